# baseline (speedup 1.0000x reference)
import functools

import jax
import jax.numpy as jnp
from jax.experimental import pallas as pl

_NP = 100000
_H = 256
_D = 128


def _gat_live(h, ss, sd, maxss, s, d, b):
    e = jax.nn.leaky_relu(ss[s] + sd[d], 0.2)
    c = jax.nn.leaky_relu(maxss + sd, 0.2)
    ex = jnp.exp(e - c[d])
    den = jax.ops.segment_sum(ex, d, num_segments=_NP)
    alpha = ex / jnp.maximum(den[d], 1e-16)
    out = jax.ops.segment_sum(h[s] * alpha[:, None], d, num_segments=_NP)
    return out + b


def _readout_body(p1_ref, p2_ref, w1_ref, w2_ref, b_ref, o_ref):
    acc = p1_ref[...] @ w1_ref[...] + p2_ref[...] @ w2_ref[...] + b_ref[0, 0]
    o_ref[...] = jax.nn.sigmoid(acc)


def kernel(x_protein, x_class, W_a_pos, as_a_pos, ad_a_pos, b_a_pos, W_a_neg, as_a_neg, ad_a_neg, b_a_neg, W_link_rel, W_link_root, b_link, W_a_ppi, as_a_ppi, ad_a_ppi, b_a_ppi, W_b_pos, as_b_pos, ad_b_pos, b_b_pos, W_b_neg, as_b_neg, ad_b_neg, b_b_neg, W_b_ppi, as_b_ppi, ad_b_ppi, b_b_ppi, W_lin, b_lin, edge_index_pos, edge_index_neg, edge_index_link, edge_index_ppi, mask):
    s = edge_index_ppi[0]
    d = edge_index_ppi[1]

    h = x_protein @ W_a_ppi
    ss = h @ as_a_ppi
    sd = h @ ad_a_ppi
    hp = jax.nn.relu(_gat_live(h, ss, sd, jnp.max(ss), s, d, b_a_ppi))

    h2 = hp @ W_b_ppi
    ss2 = h2 @ as_b_ppi
    sd2 = h2 @ ad_b_ppi
    hp2 = _gat_live(h2, ss2, sd2, jnp.max(ss2), s, d, b_b_ppi)

    p1 = hp2[mask[:, 0]]
    p2 = hp2[mask[:, 1]]
    w1 = W_lin[:_D]
    w2 = W_lin[_D:]
    out = pl.pallas_call(
        _readout_body,
        out_shape=jax.ShapeDtypeStruct((mask.shape[0], 1), jnp.float32),
    )(p1, p2, w1, w2, b_lin.reshape(1, 1))
    return out


# speedup vs baseline: 3.2163x; 3.2163x over previous
import functools

import jax
import jax.numpy as jnp
from jax import lax
from jax.experimental import pallas as pl
from jax.experimental.pallas import tpu as pltpu
from jax.experimental.pallas import tpu_sc as plsc

_NP = 100000
_E = 300000
_D = 128
_H = 256

_NCORE = 2
_NSUB = 16
_NW = _NCORE * _NSUB

_N_PAD = 100352
_E_PAD = 327680
_EROWS = _E_PAD // 128
_CR = 16
_PB = 128
_CAP = 2304


def _mm_body(prelu, x_ref, w_ref, b_ref, as_ref, ad_ref, h_ref, ss_ref,
             sd_ref, mx_ref):
    i = pl.program_id(0)
    x = x_ref[...]
    if prelu:
        x = jax.nn.relu(x + b_ref[...])
    h = x @ w_ref[...]
    h_ref[...] = h
    ss = jnp.sum(h * as_ref[...], axis=1, keepdims=True)
    sd = jnp.sum(h * ad_ref[...], axis=1, keepdims=True)
    ss_ref[...] = ss
    sd_ref[...] = sd

    @pl.when(i == 0)
    def _():
        mx_ref[...] = jnp.full((1, 1), -jnp.inf, jnp.float32)
    mx_ref[...] = jnp.maximum(mx_ref[...], jnp.max(ss).reshape(1, 1))


def _mm_scores(x, w, a_s, a_d, bias, prelu, r):
    n, kdim = x.shape
    f = w.shape[1]
    grid = (n // r,)
    return pl.pallas_call(
        functools.partial(_mm_body, prelu),
        grid=grid,
        in_specs=[
            pl.BlockSpec((r, kdim), lambda i: (i, 0)),
            pl.BlockSpec((kdim, f), lambda i: (0, 0)),
            pl.BlockSpec((1, kdim), lambda i: (0, 0)),
            pl.BlockSpec((1, f), lambda i: (0, 0)),
            pl.BlockSpec((1, f), lambda i: (0, 0)),
        ],
        out_specs=[
            pl.BlockSpec((r, f), lambda i: (i, 0)),
            pl.BlockSpec((r, 1), lambda i: (i, 0)),
            pl.BlockSpec((r, 1), lambda i: (i, 0)),
            pl.BlockSpec((1, 1), lambda i: (0, 0)),
        ],
        out_shape=[
            jax.ShapeDtypeStruct((n, f), jnp.float32),
            jax.ShapeDtypeStruct((n, 1), jnp.float32),
            jax.ShapeDtypeStruct((n, 1), jnp.float32),
            jax.ShapeDtypeStruct((1, 1), jnp.float32),
        ],
    )(x, w, bias.reshape(1, kdim), a_s.reshape(1, f), a_d.reshape(1, f))


def _readout_body(p1_ref, p2_ref, w1_ref, w2_ref, bb_ref, b_ref, o_ref):
    p1 = p1_ref[...] + bb_ref[...]
    p2 = p2_ref[...] + bb_ref[...]
    acc = p1 @ w1_ref[...] + p2 @ w2_ref[...] + b_ref[0, 0]
    o_ref[...] = jax.nn.sigmoid(acc)


def _lrelu(x):
    return jnp.where(x >= 0.0, x, 0.2 * x)


def _edge_ex_den(ss_pad, sd_pad, mx, s2, d2):
    rows_per_tile = _EROWS // _NW
    nchunk = rows_per_tile // _CR
    zslice = _N_PAD // _NSUB
    zlen = _CR * 128

    @functools.partial(
        pl.kernel,
        out_type=[
            jax.ShapeDtypeStruct((_EROWS, 128), jnp.float32),
            jax.ShapeDtypeStruct((_NCORE * _N_PAD,), jnp.float32),
        ],
        mesh=plsc.VectorSubcoreMesh(core_axis_name="c", subcore_axis_name="s"),
        scratch_types=[
            pltpu.VMEM((_CR, 128), jnp.int32),
            pltpu.VMEM((_CR, 128), jnp.int32),
            pltpu.VMEM((_CR, 128), jnp.float32),
            pltpu.VMEM((_CR, 128), jnp.float32),
            pltpu.VMEM((_CR, 128), jnp.float32),
            pltpu.VMEM((zlen,), jnp.float32),
            pltpu.VMEM((16,), jnp.float32),
            pltpu.VMEM_SHARED((_N_PAD,), jnp.float32),
            pltpu.SemaphoreType.DMA,
        ],
    )
    def k(ss_hbm, sd_hbm, mx_hbm, s_hbm, d_hbm, ex_hbm, den_hbm,
          s_v, d_v, ex_v, ssg_v, sdg_v, zz_v, mx_v, den_sh, sem):
        cid = lax.axis_index("c")
        sid = lax.axis_index("s")
        wid = sid * _NCORE + cid
        pltpu.sync_copy(mx_hbm, mx_v)
        mx_s = mx_v[pl.ds(0, 16)][0]

        def zb(i, _):
            zz_v[pl.ds(i * 16, 16)] = jnp.zeros((16,), jnp.float32)
            return ()
        lax.fori_loop(0, zlen // 16, zb, ())

        def zcopy(i, _):
            pltpu.sync_copy(zz_v, den_sh.at[pl.ds(sid * zslice + i * zlen,
                                                  zlen)])
            return ()
        lax.fori_loop(0, 3, zcopy, ())
        pltpu.sync_copy(zz_v.at[pl.ds(0, zslice - 3 * zlen)],
                        den_sh.at[pl.ds(sid * zslice + 3 * zlen,
                                        zslice - 3 * zlen)])
        plsc.subcore_barrier()

        base = wid * rows_per_tile

        def chunk_body(ci, _):
            off = base + ci * _CR
            pltpu.sync_copy(s_hbm.at[pl.ds(off, _CR)], s_v)
            pltpu.sync_copy(d_hbm.at[pl.ds(off, _CR)], d_v)

            descs = []
            for rr in range(_CR):
                descs.append(pltpu.async_copy(
                    ss_hbm.at[s_v.at[rr]], ssg_v.at[rr], sem))
                descs.append(pltpu.async_copy(
                    sd_hbm.at[d_v.at[rr]], sdg_v.at[rr], sem))
            for c in descs:
                c.wait()

            def row_body(rr, _):
                def vbody(j, _):
                    ssj = ssg_v[rr, pl.ds(j * 16, 16)]
                    sdj = sdg_v[rr, pl.ds(j * 16, 16)]
                    e = _lrelu(ssj + sdj)
                    c = _lrelu(mx_s + sdj)
                    ex_v[rr, pl.ds(j * 16, 16)] = jnp.exp(e - c)
                    return ()
                lax.fori_loop(0, 8, vbody, ())
                return ()
            lax.fori_loop(0, _CR, row_body, ())

            descs = []
            for rr in range(_CR):
                c = pltpu.make_async_copy(
                    ex_v.at[rr], den_sh.at[d_v.at[rr]], sem)
                c.start(add=True)
                descs.append(c)
            for c in descs:
                c.wait()
            pltpu.sync_copy(ex_v, ex_hbm.at[pl.ds(off, _CR)])
            return ()
        lax.fori_loop(0, nchunk, chunk_body, ())

        plsc.subcore_barrier()
        pltpu.sync_copy(den_sh.at[pl.ds(sid * zslice, zslice)],
                        den_hbm.at[pl.ds(cid * _N_PAD + sid * zslice,
                                         zslice)])

    return k(ss_pad, sd_pad, mx, s2, d2)


def _invden(den_part):
    per_tile = _N_PAD // _NW

    @functools.partial(
        pl.kernel,
        out_type=jax.ShapeDtypeStruct((_N_PAD,), jnp.float32),
        mesh=plsc.VectorSubcoreMesh(core_axis_name="c", subcore_axis_name="s"),
        scratch_types=[
            pltpu.VMEM((per_tile,), jnp.float32),
            pltpu.VMEM((per_tile,), jnp.float32),
        ],
    )
    def k(den_hbm, inv_hbm, a_v, b_v):
        cid = lax.axis_index("c")
        sid = lax.axis_index("s")
        wid = sid * _NCORE + cid
        base = wid * per_tile
        pltpu.sync_copy(den_hbm.at[pl.ds(base, per_tile)], a_v)
        pltpu.sync_copy(den_hbm.at[pl.ds(_N_PAD + base, per_tile)], b_v)

        def vbody(j, _):
            v = a_v[pl.ds(j * 16, 16)] + b_v[pl.ds(j * 16, 16)]
            a_v[pl.ds(j * 16, 16)] = 1.0 / jnp.maximum(v, 1e-16)
            return ()
        lax.fori_loop(0, per_tile // 16, vbody, ())
        pltpu.sync_copy(a_v, inv_hbm.at[pl.ds(base, per_tile)])

    return k(den_part)


_BN = 256
_NBKT = _N_PAD // _BN
_HIST = _NBKT * _NW
_HPAD = 12800


def _bucket_hist(d2):
    rows_per_tile = _EROWS // _NW
    nchunk = rows_per_tile // _CR
    zslice = _HIST // _NSUB

    @functools.partial(
        pl.kernel,
        out_type=jax.ShapeDtypeStruct((_NCORE * _HIST,), jnp.float32),
        mesh=plsc.VectorSubcoreMesh(core_axis_name="c", subcore_axis_name="s"),
        scratch_types=[
            pltpu.VMEM((_CR, 128), jnp.int32),
            pltpu.VMEM((_CR, 128), jnp.int32),
            pltpu.VMEM((784,), jnp.float32),
            pltpu.VMEM((128,), jnp.float32),
            pltpu.VMEM_SHARED((_HIST,), jnp.float32),
            pltpu.SemaphoreType.DMA,
        ],
    )
    def k(d_hbm, hist_hbm, d_v, k_v, zz_v, one_v, hist_sh, hsem):
        cid = lax.axis_index("c")
        sid = lax.axis_index("s")
        wid = sid * _NCORE + cid

        def zb(i, _):
            zz_v[pl.ds(i * 16, 16)] = jnp.zeros((16,), jnp.float32)
            one_v[pl.ds(i * 16, 16)] = jnp.ones((16,), jnp.float32)
            return ()
        lax.fori_loop(0, 8, zb, ())

        def zb2(i, _):
            zz_v[pl.ds(128 + i * 16, 16)] = jnp.zeros((16,), jnp.float32)
            return ()
        lax.fori_loop(0, (784 - 128) // 16, zb2, ())
        pltpu.sync_copy(zz_v, hist_sh.at[pl.ds(sid * zslice, zslice)])
        plsc.subcore_barrier()

        base = wid * rows_per_tile

        def chunk_body(ci, _):
            off = base + ci * _CR
            pltpu.sync_copy(d_hbm.at[pl.ds(off, _CR)], d_v)

            def rbody(rr, _):
                def vbody(j, _):
                    dv = d_v[rr, pl.ds(j * 16, 16)]
                    k_v[rr, pl.ds(j * 16, 16)] = (
                        (dv >> 8) * _NW + wid)
                    return ()
                lax.fori_loop(0, 8, vbody, ())
                return ()
            lax.fori_loop(0, _CR, rbody, ())

            descs = []
            for rr in range(_CR):
                c = pltpu.make_async_copy(one_v, hist_sh.at[k_v.at[rr]],
                                          hsem)
                c.start(add=True)
                descs.append(c)
            for c in descs:
                c.wait()
            return ()
        lax.fori_loop(0, nchunk, chunk_body, ())

        plsc.subcore_barrier()

        @pl.when(sid == 0)
        def _():
            pltpu.sync_copy(hist_sh, hist_hbm.at[pl.ds(cid * _HIST, _HIST)])

    return k(d2)


def _bucket_scan(hist):
    @functools.partial(
        pl.kernel,
        out_type=[
            jax.ShapeDtypeStruct((_HPAD,), jnp.int32),
            jax.ShapeDtypeStruct((416,), jnp.int32),
        ],
        mesh=plsc.VectorSubcoreMesh(core_axis_name="c", subcore_axis_name="s"),
        compiler_params=pltpu.CompilerParams(needs_layout_passes=False),
        scratch_types=[
            pltpu.VMEM((_HIST,), jnp.float32),
            pltpu.VMEM((_HIST,), jnp.float32),
            pltpu.VMEM((_HPAD,), jnp.int32),
            pltpu.VMEM((416,), jnp.int32),
        ],
    )
    def k(hist_hbm, offs_hbm, st_hbm, a_v, b_v, o_v, st_v):
        cid = lax.axis_index("c")
        sid = lax.axis_index("s")

        @pl.when((cid == 0) & (sid == 0))
        def _():
            pltpu.sync_copy(hist_hbm.at[pl.ds(0, _HIST)], a_v)
            pltpu.sync_copy(hist_hbm.at[pl.ds(_HIST, _HIST)], b_v)

            def sc(i, carry):
                v = (a_v[pl.ds(i * 16, 16)]
                     + b_v[pl.ds(i * 16, 16)]).astype(jnp.int32)
                run = plsc.cumsum(v)
                o_v[pl.ds(i * 16, 16)] = carry + run - v
                return carry + run[15]
            lax.fori_loop(0, _HIST // 16, sc, jnp.int32(0))

            def pz(i, _):
                o_v[pl.ds(_HIST + i * 16, 16)] = jnp.full(
                    (16,), _E_PAD, jnp.int32)
                return ()
            lax.fori_loop(0, (_HPAD - _HIST) // 16, pz, ())

            def stb(h, _):
                bno = h * 16 + lax.iota(jnp.int32, 16)
                idx = jnp.minimum(bno * _NW, _HPAD - 16)
                g = plsc.load_gather(o_v, [idx])
                st_v[pl.ds(h * 16, 16)] = jnp.where(bno >= _NBKT,
                                                    _E_PAD, g)
                return ()
            lax.fori_loop(0, 26, stb, ())
            pltpu.sync_copy(o_v, offs_hbm)
            pltpu.sync_copy(st_v, st_hbm)

    return k(hist)


def _bucket_place(s2, d2, offs):
    rows_per_tile = _EROWS // _NW
    nflush = rows_per_tile // 8

    @functools.partial(
        pl.kernel,
        out_type=[
            jax.ShapeDtypeStruct((_E_PAD,), jnp.int32),
            jax.ShapeDtypeStruct((_E_PAD,), jnp.int32),
        ],
        mesh=plsc.VectorSubcoreMesh(core_axis_name="c", subcore_axis_name="s"),
        compiler_params=pltpu.CompilerParams(needs_layout_passes=False),
        scratch_types=[
            pltpu.VMEM((8, 128), jnp.int32),
            pltpu.VMEM((8, 128), jnp.int32),
            pltpu.VMEM((8, 128), jnp.int32),
            pltpu.VMEM((_HPAD,), jnp.int32),
            pltpu.VMEM((48,), jnp.int32),
            pltpu.SemaphoreType.DMA,
        ],
    )
    def k(s_hbm, d_hbm, offs_hbm, sg_hbm, dg_hbm,
          s_v, d_v, p_v, offs_v, win_v, sem):
        cid = lax.axis_index("c")
        sid = lax.axis_index("s")
        wid = sid * _NCORE + cid
        pltpu.sync_copy(offs_hbm, offs_v)
        win_v[pl.ds(0, 16)] = jnp.full((16,), -1, jnp.int32)
        win_v[pl.ds(32, 16)] = jnp.full((16,), -2, jnp.int32)

        base = wid * rows_per_tile

        def flush_body(fi, _):
            off = base + fi * 8
            pltpu.sync_copy(s_hbm.at[pl.ds(off, 8)], s_v)
            pltpu.sync_copy(d_hbm.at[pl.ds(off, 8)], d_v)

            def rbody(rr, _):
                def vbody(j, _):
                    dv = d_v[rr, pl.ds(j * 16, 16)]
                    bk = dv >> 8
                    win_v[pl.ds(16, 16)] = bk
                    rf = jnp.zeros((16,), jnp.int32)
                    rb = jnp.zeros((16,), jnp.int32)
                    for sh in range(1, 16):
                        wf = win_v[pl.ds(16 - sh, 16)]
                        wb = win_v[pl.ds(16 + sh, 16)]
                        rf = rf + (wf == bk).astype(jnp.int32)
                        rb = rb + (wb == bk).astype(jnp.int32)
                    key = bk * _NW + wid
                    bs = plsc.load_gather(offs_v, [key])
                    p_v[rr, pl.ds(j * 16, 16)] = bs + rf
                    plsc.store_scatter(offs_v, [key], bs + rf + rb + 1)
                    return ()
                lax.fori_loop(0, 8, vbody, ())
                return ()
            lax.fori_loop(0, 8, rbody, ())

            descs = []
            for rr in range(8):
                descs.append(pltpu.async_copy(
                    s_v.at[rr], sg_hbm.at[p_v.at[rr]], sem))
                descs.append(pltpu.async_copy(
                    d_v.at[rr], dg_hbm.at[p_v.at[rr]], sem))
            for c in descs:
                c.wait()
            return ()
        lax.fori_loop(0, nflush, flush_body, ())

    return k(s2, d2, offs)


def _weighted_scatter(h, exg, invd, sg, dg, starts, feat):
    pb = 112

    @functools.partial(
        pl.kernel,
        out_type=jax.ShapeDtypeStruct((_N_PAD, feat), jnp.float32),
        mesh=plsc.VectorSubcoreMesh(core_axis_name="c", subcore_axis_name="s"),
        compiler_params=pltpu.CompilerParams(needs_layout_passes=False),
        scratch_types=[
            [pltpu.VMEM((pb,), jnp.int32)] * 2,
            [pltpu.VMEM((pb,), jnp.int32)] * 2,
            [pltpu.VMEM((pb,), jnp.float32)] * 2,
            pltpu.VMEM((_BN,), jnp.float32),
            [pltpu.VMEM((pb, feat), jnp.float32)] * 2,
            pltpu.VMEM((pb,), jnp.float32),
            pltpu.VMEM((pb,), jnp.int32),
            pltpu.VMEM((_BN, feat), jnp.float32),
            pltpu.VMEM((416,), jnp.int32),
            pltpu.SemaphoreType.DMA,
        ],
    )
    def k(h_hbm, ex_hbm, inv_hbm, s_hbm, d_hbm, st_hbm, out_hbm,
          s_b, d_b, ex_b, ivd_v, rows_b, al_v, dl_v, acc_v, st_v,
          gsem):
        cid = lax.axis_index("c")
        sid = lax.axis_index("s")
        wid = sid * _NCORE + cid
        pltpu.sync_copy(st_hbm, st_v)
        nb_w = (_NBKT - wid + _NW - 1) // _NW

        def issue(kk, p, start8):
            bo = start8 + kk * pb
            pltpu.sync_copy(s_hbm.at[pl.ds(bo, pb)], s_b[p])
            pltpu.sync_copy(d_hbm.at[pl.ds(bo, pb)], d_b[p])
            pltpu.sync_copy(ex_hbm.at[pl.ds(bo, pb)], ex_b[p])
            pltpu.async_copy(h_hbm.at[s_b[p]], rows_b[p], gsem)

        def waitp(p):
            pltpu.make_async_copy(h_hbm.at[s_b[p]], rows_b[p], gsem).wait()

        def process(kk, p, lo, eb0, eb1, start8):
            bo = start8 + kk * pb

            def ab(j, _):
                ge = bo + j * 16 + lax.iota(jnp.int32, 16)
                valid = (ge >= eb0) & (ge < eb1)
                dv = d_b[p][pl.ds(j * 16, 16)]
                dl = jnp.clip(dv - lo, 0, _BN - 1)
                dl_v[pl.ds(j * 16, 16)] = dl
                iv = plsc.load_gather(ivd_v, [dl])
                al_v[pl.ds(j * 16, 16)] = jnp.where(
                    valid,
                    ex_b[p][pl.ds(j * 16, 16)] * iv,
                    0.0)
                return ()
            lax.fori_loop(0, pb // 16, ab, ())

            def rb(g, _):
                av = al_v[pl.ds(g * 16, 16)]
                dlv = dl_v[pl.ds(g * 16, 16)]
                for t in range(16):
                    a = av[t]
                    dli = dlv[t]
                    i = g * 16 + t
                    for j in range(feat // 16):
                        acc_v[dli, pl.ds(j * 16, 16)] = (
                            acc_v[dli, pl.ds(j * 16, 16)]
                            + rows_b[p][i, pl.ds(j * 16, 16)] * a)
                return ()
            lax.fori_loop(0, pb // 16, rb, ())

        def bucket_body(bi, _):
            b = wid + bi * _NW
            lo = b * _BN
            sv = st_v[pl.ds(b, 16)]
            eb0 = sv[0]
            eb1 = sv[1]
            pltpu.sync_copy(inv_hbm.at[pl.ds(lo, _BN)], ivd_v)

            def za(i, _):
                for j in range(feat // 16):
                    acc_v[i, pl.ds(j * 16, 16)] = jnp.zeros(
                        (16,), jnp.float32)
                return ()
            lax.fori_loop(0, _BN, za, ())

            start8 = (eb0 // 8) * 8
            nblk = (eb1 - start8 + pb - 1) // pb

            @pl.when(nblk > 0)
            def _():
                issue(0, 0, start8)

                def pair(m, _):
                    k0 = 2 * m
                    k1 = 2 * m + 1
                    waitp(0)

                    @pl.when(k1 < nblk)
                    def _():
                        issue(k1, 1, start8)
                    process(k0, 0, lo, eb0, eb1, start8)

                    @pl.when(k1 < nblk)
                    def _():
                        waitp(1)

                        @pl.when(k1 + 1 < nblk)
                        def _():
                            issue(k1 + 1, 0, start8)
                        process(k1, 1, lo, eb0, eb1, start8)
                    return ()
                lax.fori_loop(0, (nblk + 1) // 2, pair, ())

            pltpu.sync_copy(acc_v, out_hbm.at[pl.ds(lo, _BN)])
            return ()
        lax.fori_loop(0, nb_w, bucket_body, ())

    return k(h, exg, invd, sg, dg, starts)


def _pair_gather(hp2, idx_flat, npairs):
    per_tile = 2 * npairs // _NW

    @functools.partial(
        pl.kernel,
        out_type=jax.ShapeDtypeStruct((2 * npairs, _D), jnp.float32),
        mesh=plsc.VectorSubcoreMesh(core_axis_name="c", subcore_axis_name="s"),
        scratch_types=[
            pltpu.VMEM((per_tile,), jnp.int32),
            pltpu.VMEM((per_tile, _D), jnp.float32),
            pltpu.SemaphoreType.DMA,
        ],
    )
    def k(h_hbm, idx_hbm, out_hbm, idx_v, rows_v, sem):
        cid = lax.axis_index("c")
        sid = lax.axis_index("s")
        wid = sid * _NCORE + cid
        base = wid * per_tile
        pltpu.sync_copy(idx_hbm.at[pl.ds(base, per_tile)], idx_v)

        def gb(i, _):
            pltpu.async_copy(
                h_hbm.at[idx_v.at[pl.ds(i * 128, 128)]],
                rows_v.at[pl.ds(i * 128, 128)], sem).wait()
            return ()
        lax.fori_loop(0, per_tile // 128, gb, ())
        pltpu.sync_copy(rows_v, out_hbm.at[pl.ds(base, per_tile)])

    return k(hp2, idx_flat)


def _col_perm(feat):
    p = []
    for g in range(feat // 32):
        p.extend(32 * g + 2 * t for t in range(16))
        p.extend(32 * g + 2 * t + 1 for t in range(16))
    return jnp.array(p, jnp.int32)


def _gat_layer(x, w, a_s, a_d, bias_in, prelu, r, sg2, dg2, sg1, dg1,
               starts, feat):
    h, ss_c, sd_c, mx = _mm_scores(x, w, a_s, a_d, bias_in, prelu, r)
    pad = _N_PAD - x.shape[0]
    ss_pad = jnp.pad(ss_c.reshape(-1), (0, pad))
    sd_pad = jnp.pad(sd_c.reshape(-1), (0, pad))
    mx16 = jnp.broadcast_to(mx[0], (16,))
    ex2, den = _edge_ex_den(ss_pad, sd_pad, mx16, sg2, dg2)
    invd = _invden(den)
    return _weighted_scatter(h, ex2.reshape(-1), invd, sg1, dg1, starts,
                             feat)


def kernel(x_protein, x_class, W_a_pos, as_a_pos, ad_a_pos, b_a_pos, W_a_neg, as_a_neg, ad_a_neg, b_a_neg, W_link_rel, W_link_root, b_link, W_a_ppi, as_a_ppi, ad_a_ppi, b_a_ppi, W_b_pos, as_b_pos, ad_b_pos, b_b_pos, W_b_neg, as_b_neg, ad_b_neg, b_b_neg, W_b_ppi, as_b_ppi, ad_b_ppi, b_b_ppi, W_lin, b_lin, edge_index_pos, edge_index_neg, edge_index_link, edge_index_ppi, mask):
    epad = _E_PAD - _E
    s2 = jnp.concatenate([edge_index_ppi[0].astype(jnp.int32),
                          jnp.zeros((epad,), jnp.int32)]).reshape(_EROWS, 128)
    d2 = jnp.concatenate([edge_index_ppi[1].astype(jnp.int32),
                          jnp.full((epad,), _NP, jnp.int32)]
                         ).reshape(_EROWS, 128)

    hist = _bucket_hist(d2)
    offs, starts = _bucket_scan(hist)
    sg, dg = _bucket_place(s2, d2, offs)
    sg2 = sg.reshape(_EROWS, 128)
    dg2 = dg.reshape(_EROWS, 128)

    out_a = _gat_layer(x_protein, W_a_ppi, as_a_ppi, ad_a_ppi,
                       jnp.zeros((_D,), jnp.float32), False, 1000,
                       sg2, dg2, sg, dg, starts, _H)

    out_b = _gat_layer(out_a, W_b_ppi, as_b_ppi, ad_b_ppi,
                       b_a_ppi, True, 784, sg2, dg2, sg, dg, starts, _D)

    npairs = mask.shape[0]
    idx_flat = mask.T.reshape(-1).astype(jnp.int32)
    rows = _pair_gather(out_b, idx_flat, npairs)
    p1 = rows[:npairs]
    p2 = rows[npairs:]
    out = pl.pallas_call(
        _readout_body,
        out_shape=jax.ShapeDtypeStruct((npairs, 1), jnp.float32),
    )(p1, p2, W_lin[:_D], W_lin[_D:], b_b_ppi.reshape(1, _D),
      b_lin.reshape(1, 1))
    return out


# speedup vs baseline: 3.2177x; 1.0004x over previous
import functools

import jax
import jax.numpy as jnp
from jax import lax
from jax.experimental import pallas as pl
from jax.experimental.pallas import tpu as pltpu
from jax.experimental.pallas import tpu_sc as plsc

_NP = 100000
_E = 300000
_D = 128
_H = 256

_NCORE = 2
_NSUB = 16
_NW = _NCORE * _NSUB

_N_PAD = 100352
_E_PAD = 327680
_EROWS = _E_PAD // 128
_CR = 16
_PB = 128


def _mm_body(prelu, x_ref, w_ref, b_ref, as_ref, ad_ref, h_ref, ss_ref,
             sd_ref, mx_ref):
    i = pl.program_id(0)
    x = x_ref[...]
    if prelu:
        x = jax.nn.relu(x + b_ref[...])
    h = x @ w_ref[...]
    h_ref[...] = h
    ss = jnp.sum(h * as_ref[...], axis=1, keepdims=True)
    sd = jnp.sum(h * ad_ref[...], axis=1, keepdims=True)
    ss_ref[...] = ss
    sd_ref[...] = sd

    @pl.when(i == 0)
    def _():
        mx_ref[...] = jnp.full((1, 1), -jnp.inf, jnp.float32)
    mx_ref[...] = jnp.maximum(mx_ref[...], jnp.max(ss).reshape(1, 1))


def _mm_scores(x, w, a_s, a_d, bias, prelu, r):
    n, kdim = x.shape
    f = w.shape[1]
    grid = (n // r,)
    return pl.pallas_call(
        functools.partial(_mm_body, prelu),
        grid=grid,
        in_specs=[
            pl.BlockSpec((r, kdim), lambda i: (i, 0)),
            pl.BlockSpec((kdim, f), lambda i: (0, 0)),
            pl.BlockSpec((1, kdim), lambda i: (0, 0)),
            pl.BlockSpec((1, f), lambda i: (0, 0)),
            pl.BlockSpec((1, f), lambda i: (0, 0)),
        ],
        out_specs=[
            pl.BlockSpec((r, f), lambda i: (i, 0)),
            pl.BlockSpec((r, 1), lambda i: (i, 0)),
            pl.BlockSpec((r, 1), lambda i: (i, 0)),
            pl.BlockSpec((1, 1), lambda i: (0, 0)),
        ],
        out_shape=[
            jax.ShapeDtypeStruct((n, f), jnp.float32),
            jax.ShapeDtypeStruct((n, 1), jnp.float32),
            jax.ShapeDtypeStruct((n, 1), jnp.float32),
            jax.ShapeDtypeStruct((1, 1), jnp.float32),
        ],
    )(x, w, bias.reshape(1, kdim), a_s.reshape(1, f), a_d.reshape(1, f))


def _readout_body(p1_ref, p2_ref, w1_ref, w2_ref, bb_ref, b_ref, o_ref):
    p1 = p1_ref[...] + bb_ref[...]
    p2 = p2_ref[...] + bb_ref[...]
    acc = p1 @ w1_ref[...] + p2 @ w2_ref[...] + b_ref[0, 0]
    o_ref[...] = jax.nn.sigmoid(acc)


def _lrelu(x):
    return jnp.where(x >= 0.0, x, 0.2 * x)


def _edge_ex_den(ss_pad, sd_pad, mx, s2, d2):
    rows_per_tile = _EROWS // _NW
    nchunk = rows_per_tile // _CR
    zslice = _N_PAD // _NSUB
    zlen = _CR * 128

    @functools.partial(
        pl.kernel,
        out_type=[
            jax.ShapeDtypeStruct((_EROWS, 128), jnp.float32),
            jax.ShapeDtypeStruct((_NCORE * _N_PAD,), jnp.float32),
        ],
        mesh=plsc.VectorSubcoreMesh(core_axis_name="c", subcore_axis_name="s"),
        scratch_types=[
            pltpu.VMEM((_CR, 128), jnp.int32),
            pltpu.VMEM((_CR, 128), jnp.int32),
            pltpu.VMEM((_CR, 128), jnp.float32),
            pltpu.VMEM((_CR, 128), jnp.float32),
            pltpu.VMEM((_CR, 128), jnp.float32),
            pltpu.VMEM((zlen,), jnp.float32),
            pltpu.VMEM((16,), jnp.float32),
            pltpu.VMEM_SHARED((_N_PAD,), jnp.float32),
            pltpu.SemaphoreType.DMA,
        ],
    )
    def k(ss_hbm, sd_hbm, mx_hbm, s_hbm, d_hbm, ex_hbm, den_hbm,
          s_v, d_v, ex_v, ssg_v, sdg_v, zz_v, mx_v, den_sh, sem):
        cid = lax.axis_index("c")
        sid = lax.axis_index("s")
        wid = sid * _NCORE + cid
        pltpu.sync_copy(mx_hbm, mx_v)
        mx_s = mx_v[pl.ds(0, 16)][0]

        def zb(i, _):
            zz_v[pl.ds(i * 16, 16)] = jnp.zeros((16,), jnp.float32)
            return ()
        lax.fori_loop(0, zlen // 16, zb, ())

        def zcopy(i, _):
            pltpu.sync_copy(zz_v, den_sh.at[pl.ds(sid * zslice + i * zlen,
                                                  zlen)])
            return ()
        lax.fori_loop(0, 3, zcopy, ())
        pltpu.sync_copy(zz_v.at[pl.ds(0, zslice - 3 * zlen)],
                        den_sh.at[pl.ds(sid * zslice + 3 * zlen,
                                        zslice - 3 * zlen)])
        plsc.subcore_barrier()

        base = wid * rows_per_tile

        def chunk_body(ci, _):
            off = base + ci * _CR
            pltpu.sync_copy(s_hbm.at[pl.ds(off, _CR)], s_v)
            pltpu.sync_copy(d_hbm.at[pl.ds(off, _CR)], d_v)

            descs = []
            for rr in range(_CR):
                descs.append(pltpu.async_copy(
                    ss_hbm.at[s_v.at[rr]], ssg_v.at[rr], sem))
                descs.append(pltpu.async_copy(
                    sd_hbm.at[d_v.at[rr]], sdg_v.at[rr], sem))
            for c in descs:
                c.wait()

            def row_body(rr, _):
                def vbody(j, _):
                    ssj = ssg_v[rr, pl.ds(j * 16, 16)]
                    sdj = sdg_v[rr, pl.ds(j * 16, 16)]
                    e = _lrelu(ssj + sdj)
                    c = _lrelu(mx_s + sdj)
                    ex_v[rr, pl.ds(j * 16, 16)] = jnp.exp(e - c)
                    return ()
                lax.fori_loop(0, 8, vbody, ())
                return ()
            lax.fori_loop(0, _CR, row_body, ())

            descs = []
            for rr in range(_CR):
                c = pltpu.make_async_copy(
                    ex_v.at[rr], den_sh.at[d_v.at[rr]], sem)
                c.start(add=True)
                descs.append(c)
            for c in descs:
                c.wait()
            pltpu.sync_copy(ex_v, ex_hbm.at[pl.ds(off, _CR)])
            return ()
        lax.fori_loop(0, nchunk, chunk_body, ())

        plsc.subcore_barrier()
        pltpu.sync_copy(den_sh.at[pl.ds(sid * zslice, zslice)],
                        den_hbm.at[pl.ds(cid * _N_PAD + sid * zslice,
                                         zslice)])

    return k(ss_pad, sd_pad, mx, s2, d2)


def _invden(den_part):
    per_tile = _N_PAD // _NW

    @functools.partial(
        pl.kernel,
        out_type=jax.ShapeDtypeStruct((_N_PAD,), jnp.float32),
        mesh=plsc.VectorSubcoreMesh(core_axis_name="c", subcore_axis_name="s"),
        scratch_types=[
            pltpu.VMEM((per_tile,), jnp.float32),
            pltpu.VMEM((per_tile,), jnp.float32),
        ],
    )
    def k(den_hbm, inv_hbm, a_v, b_v):
        cid = lax.axis_index("c")
        sid = lax.axis_index("s")
        wid = sid * _NCORE + cid
        base = wid * per_tile
        pltpu.sync_copy(den_hbm.at[pl.ds(base, per_tile)], a_v)
        pltpu.sync_copy(den_hbm.at[pl.ds(_N_PAD + base, per_tile)], b_v)

        def vbody(j, _):
            v = a_v[pl.ds(j * 16, 16)] + b_v[pl.ds(j * 16, 16)]
            a_v[pl.ds(j * 16, 16)] = 1.0 / jnp.maximum(v, 1e-16)
            return ()
        lax.fori_loop(0, per_tile // 16, vbody, ())
        pltpu.sync_copy(a_v, inv_hbm.at[pl.ds(base, per_tile)])

    return k(den_part)


_BN = 256
_NBKT = _N_PAD // _BN
_HIST = _NBKT * _NW
_HPAD = 12800


def _bucket_hist(d2):
    rows_per_tile = _EROWS // _NW
    nchunk = rows_per_tile // _CR
    zslice = _HIST // _NSUB

    @functools.partial(
        pl.kernel,
        out_type=jax.ShapeDtypeStruct((_NCORE * _HIST,), jnp.float32),
        mesh=plsc.VectorSubcoreMesh(core_axis_name="c", subcore_axis_name="s"),
        scratch_types=[
            pltpu.VMEM((_CR, 128), jnp.int32),
            pltpu.VMEM((_CR, 128), jnp.int32),
            pltpu.VMEM((784,), jnp.float32),
            pltpu.VMEM((128,), jnp.float32),
            pltpu.VMEM_SHARED((_HIST,), jnp.float32),
            pltpu.SemaphoreType.DMA,
        ],
    )
    def k(d_hbm, hist_hbm, d_v, k_v, zz_v, one_v, hist_sh, hsem):
        cid = lax.axis_index("c")
        sid = lax.axis_index("s")
        wid = sid * _NCORE + cid

        def zb(i, _):
            zz_v[pl.ds(i * 16, 16)] = jnp.zeros((16,), jnp.float32)
            one_v[pl.ds(i * 16, 16)] = jnp.ones((16,), jnp.float32)
            return ()
        lax.fori_loop(0, 8, zb, ())

        def zb2(i, _):
            zz_v[pl.ds(128 + i * 16, 16)] = jnp.zeros((16,), jnp.float32)
            return ()
        lax.fori_loop(0, (784 - 128) // 16, zb2, ())
        pltpu.sync_copy(zz_v, hist_sh.at[pl.ds(sid * zslice, zslice)])
        plsc.subcore_barrier()

        base = wid * rows_per_tile

        def chunk_body(ci, _):
            off = base + ci * _CR
            pltpu.sync_copy(d_hbm.at[pl.ds(off, _CR)], d_v)

            def rbody(rr, _):
                def vbody(j, _):
                    dv = d_v[rr, pl.ds(j * 16, 16)]
                    k_v[rr, pl.ds(j * 16, 16)] = (
                        (dv >> 8) * _NW + wid)
                    return ()
                lax.fori_loop(0, 8, vbody, ())
                return ()
            lax.fori_loop(0, _CR, rbody, ())

            descs = []
            for rr in range(_CR):
                c = pltpu.make_async_copy(one_v, hist_sh.at[k_v.at[rr]],
                                          hsem)
                c.start(add=True)
                descs.append(c)
            for c in descs:
                c.wait()
            return ()
        lax.fori_loop(0, nchunk, chunk_body, ())

        plsc.subcore_barrier()

        @pl.when(sid == 0)
        def _():
            pltpu.sync_copy(hist_sh, hist_hbm.at[pl.ds(cid * _HIST, _HIST)])

    return k(d2)


def _bucket_scan(hist):
    @functools.partial(
        pl.kernel,
        out_type=[
            jax.ShapeDtypeStruct((_HPAD,), jnp.int32),
            jax.ShapeDtypeStruct((416,), jnp.int32),
        ],
        mesh=plsc.VectorSubcoreMesh(core_axis_name="c", subcore_axis_name="s"),
        compiler_params=pltpu.CompilerParams(needs_layout_passes=False),
        scratch_types=[
            pltpu.VMEM((_HIST,), jnp.float32),
            pltpu.VMEM((_HIST,), jnp.float32),
            pltpu.VMEM((_HPAD,), jnp.int32),
            pltpu.VMEM((416,), jnp.int32),
        ],
    )
    def k(hist_hbm, offs_hbm, st_hbm, a_v, b_v, o_v, st_v):
        cid = lax.axis_index("c")
        sid = lax.axis_index("s")

        @pl.when((cid == 0) & (sid == 0))
        def _():
            pltpu.sync_copy(hist_hbm.at[pl.ds(0, _HIST)], a_v)
            pltpu.sync_copy(hist_hbm.at[pl.ds(_HIST, _HIST)], b_v)

            def sc(i, carry):
                v = (a_v[pl.ds(i * 16, 16)]
                     + b_v[pl.ds(i * 16, 16)]).astype(jnp.int32)
                run = plsc.cumsum(v)
                o_v[pl.ds(i * 16, 16)] = carry + run - v
                return carry + run[15]
            lax.fori_loop(0, _HIST // 16, sc, jnp.int32(0))

            def pz(i, _):
                o_v[pl.ds(_HIST + i * 16, 16)] = jnp.full(
                    (16,), _E_PAD, jnp.int32)
                return ()
            lax.fori_loop(0, (_HPAD - _HIST) // 16, pz, ())

            def stb(h, _):
                bno = h * 16 + lax.iota(jnp.int32, 16)
                idx = jnp.minimum(bno * _NW, _HPAD - 16)
                g = plsc.load_gather(o_v, [idx])
                st_v[pl.ds(h * 16, 16)] = jnp.where(bno >= _NBKT,
                                                    _E_PAD, g)
                return ()
            lax.fori_loop(0, 26, stb, ())
            pltpu.sync_copy(o_v, offs_hbm)
            pltpu.sync_copy(st_v, st_hbm)

    return k(hist)


def _bucket_place(s2, d2, offs):
    rows_per_tile = _EROWS // _NW
    nflush = rows_per_tile // 8

    @functools.partial(
        pl.kernel,
        out_type=[
            jax.ShapeDtypeStruct((_E_PAD,), jnp.int32),
            jax.ShapeDtypeStruct((_E_PAD,), jnp.int32),
        ],
        mesh=plsc.VectorSubcoreMesh(core_axis_name="c", subcore_axis_name="s"),
        compiler_params=pltpu.CompilerParams(needs_layout_passes=False),
        scratch_types=[
            pltpu.VMEM((8, 128), jnp.int32),
            pltpu.VMEM((8, 128), jnp.int32),
            pltpu.VMEM((8, 128), jnp.int32),
            pltpu.VMEM((_HPAD,), jnp.int32),
            pltpu.VMEM((48,), jnp.int32),
            pltpu.SemaphoreType.DMA,
        ],
    )
    def k(s_hbm, d_hbm, offs_hbm, sg_hbm, dg_hbm,
          s_v, d_v, p_v, offs_v, win_v, sem):
        cid = lax.axis_index("c")
        sid = lax.axis_index("s")
        wid = sid * _NCORE + cid
        pltpu.sync_copy(offs_hbm, offs_v)
        win_v[pl.ds(0, 16)] = jnp.full((16,), -1, jnp.int32)
        win_v[pl.ds(32, 16)] = jnp.full((16,), -2, jnp.int32)

        base = wid * rows_per_tile

        def flush_body(fi, _):
            off = base + fi * 8
            pltpu.sync_copy(s_hbm.at[pl.ds(off, 8)], s_v)
            pltpu.sync_copy(d_hbm.at[pl.ds(off, 8)], d_v)

            def rbody(rr, _):
                def vbody(j, _):
                    dv = d_v[rr, pl.ds(j * 16, 16)]
                    bk = dv >> 8
                    win_v[pl.ds(16, 16)] = bk
                    rf = jnp.zeros((16,), jnp.int32)
                    rb = jnp.zeros((16,), jnp.int32)
                    for sh in range(1, 16):
                        wf = win_v[pl.ds(16 - sh, 16)]
                        wb = win_v[pl.ds(16 + sh, 16)]
                        rf = rf + (wf == bk).astype(jnp.int32)
                        rb = rb + (wb == bk).astype(jnp.int32)
                    key = bk * _NW + wid
                    bs = plsc.load_gather(offs_v, [key])
                    p_v[rr, pl.ds(j * 16, 16)] = bs + rf
                    plsc.store_scatter(offs_v, [key], bs + rf + rb + 1)
                    return ()
                lax.fori_loop(0, 8, vbody, ())
                return ()
            lax.fori_loop(0, 8, rbody, ())

            descs = []
            for rr in range(8):
                descs.append(pltpu.async_copy(
                    s_v.at[rr], sg_hbm.at[p_v.at[rr]], sem))
                descs.append(pltpu.async_copy(
                    d_v.at[rr], dg_hbm.at[p_v.at[rr]], sem))
            for c in descs:
                c.wait()
            return ()
        lax.fori_loop(0, nflush, flush_body, ())

    return k(s2, d2, offs)


def _weighted_scatter(h, exg, invd, sg, dg, starts, feat):
    pb = 112

    @functools.partial(
        pl.kernel,
        out_type=jax.ShapeDtypeStruct((_N_PAD, feat), jnp.float32),
        mesh=plsc.VectorSubcoreMesh(core_axis_name="c", subcore_axis_name="s"),
        compiler_params=pltpu.CompilerParams(needs_layout_passes=False),
        scratch_types=[
            [pltpu.VMEM((pb,), jnp.int32)] * 2,
            [pltpu.VMEM((pb,), jnp.int32)] * 2,
            [pltpu.VMEM((pb,), jnp.float32)] * 2,
            pltpu.VMEM((_BN,), jnp.float32),
            [pltpu.VMEM((pb, feat), jnp.float32)] * 2,
            pltpu.VMEM((pb,), jnp.float32),
            pltpu.VMEM((pb,), jnp.int32),
            pltpu.VMEM((_BN, feat), jnp.float32),
            pltpu.VMEM((416,), jnp.int32),
            pltpu.SemaphoreType.DMA,
        ],
    )
    def k(h_hbm, ex_hbm, inv_hbm, s_hbm, d_hbm, st_hbm, out_hbm,
          s_b, d_b, ex_b, ivd_v, rows_b, al_v, dl_v, acc_v, st_v,
          gsem):
        cid = lax.axis_index("c")
        sid = lax.axis_index("s")
        wid = sid * _NCORE + cid
        pltpu.sync_copy(st_hbm, st_v)
        nb_w = (_NBKT - wid + _NW - 1) // _NW

        def issue(kk, p, start8):
            bo = start8 + kk * pb
            pltpu.sync_copy(s_hbm.at[pl.ds(bo, pb)], s_b[p])
            pltpu.sync_copy(d_hbm.at[pl.ds(bo, pb)], d_b[p])
            pltpu.sync_copy(ex_hbm.at[pl.ds(bo, pb)], ex_b[p])
            pltpu.async_copy(h_hbm.at[s_b[p]], rows_b[p], gsem)

        def waitp(p):
            pltpu.make_async_copy(h_hbm.at[s_b[p]], rows_b[p], gsem).wait()

        def process(kk, p, lo, eb0, eb1, start8):
            bo = start8 + kk * pb

            def ab(j, _):
                ge = bo + j * 16 + lax.iota(jnp.int32, 16)
                valid = (ge >= eb0) & (ge < eb1)
                dv = d_b[p][pl.ds(j * 16, 16)]
                dl = jnp.clip(dv - lo, 0, _BN - 1)
                dl_v[pl.ds(j * 16, 16)] = dl
                iv = plsc.load_gather(ivd_v, [dl])
                al_v[pl.ds(j * 16, 16)] = jnp.where(
                    valid,
                    ex_b[p][pl.ds(j * 16, 16)] * iv,
                    0.0)
                return ()
            lax.fori_loop(0, pb // 16, ab, ())

            def rb(g, _):
                av = al_v[pl.ds(g * 16, 16)]
                dlv = dl_v[pl.ds(g * 16, 16)]
                for t in range(16):
                    a = av[t]
                    dli = dlv[t]
                    i = g * 16 + t
                    for j in range(feat // 16):
                        acc_v[dli, pl.ds(j * 16, 16)] = (
                            acc_v[dli, pl.ds(j * 16, 16)]
                            + rows_b[p][i, pl.ds(j * 16, 16)] * a)
                return ()
            lax.fori_loop(0, pb // 16, rb, ())

        def bucket_body(bi, _):
            b = wid + bi * _NW
            lo = b * _BN
            sv = st_v[pl.ds(b, 16)]
            eb0 = sv[0]
            eb1 = sv[1]
            pltpu.sync_copy(inv_hbm.at[pl.ds(lo, _BN)], ivd_v)

            def za(i, _):
                for j in range(feat // 16):
                    acc_v[i, pl.ds(j * 16, 16)] = jnp.zeros(
                        (16,), jnp.float32)
                return ()
            lax.fori_loop(0, _BN, za, ())

            start8 = (eb0 // 8) * 8
            nblk = (eb1 - start8 + pb - 1) // pb

            @pl.when(nblk > 0)
            def _():
                issue(0, 0, start8)

                def pair(m, _):
                    k0 = 2 * m
                    k1 = 2 * m + 1
                    waitp(0)

                    @pl.when(k1 < nblk)
                    def _():
                        issue(k1, 1, start8)
                    process(k0, 0, lo, eb0, eb1, start8)

                    @pl.when(k1 < nblk)
                    def _():
                        waitp(1)

                        @pl.when(k1 + 1 < nblk)
                        def _():
                            issue(k1 + 1, 0, start8)
                        process(k1, 1, lo, eb0, eb1, start8)
                    return ()
                lax.fori_loop(0, (nblk + 1) // 2, pair, ())

            pltpu.sync_copy(acc_v, out_hbm.at[pl.ds(lo, _BN)])
            return ()
        lax.fori_loop(0, nb_w, bucket_body, ())

    return k(h, exg, invd, sg, dg, starts)


def _pair_gather(hp2, idx_flat, npairs):
    per_tile = 2 * npairs // _NW

    @functools.partial(
        pl.kernel,
        out_type=jax.ShapeDtypeStruct((2 * npairs, _D), jnp.float32),
        mesh=plsc.VectorSubcoreMesh(core_axis_name="c", subcore_axis_name="s"),
        scratch_types=[
            pltpu.VMEM((per_tile,), jnp.int32),
            pltpu.VMEM((per_tile, _D), jnp.float32),
            pltpu.SemaphoreType.DMA,
        ],
    )
    def k(h_hbm, idx_hbm, out_hbm, idx_v, rows_v, sem):
        cid = lax.axis_index("c")
        sid = lax.axis_index("s")
        wid = sid * _NCORE + cid
        base = wid * per_tile
        pltpu.sync_copy(idx_hbm.at[pl.ds(base, per_tile)], idx_v)

        def gb(i, _):
            pltpu.async_copy(
                h_hbm.at[idx_v.at[pl.ds(i * 128, 128)]],
                rows_v.at[pl.ds(i * 128, 128)], sem).wait()
            return ()
        lax.fori_loop(0, per_tile // 128, gb, ())
        pltpu.sync_copy(rows_v, out_hbm.at[pl.ds(base, per_tile)])

    return k(hp2, idx_flat)


def _col_perm(feat):
    p = []
    for g in range(feat // 32):
        p.extend(32 * g + 2 * t for t in range(16))
        p.extend(32 * g + 2 * t + 1 for t in range(16))
    return jnp.array(p, jnp.int32)


def _gat_layer(x, w, a_s, a_d, bias_in, prelu, r, sg2, dg2, sg1, dg1,
               starts, feat):
    h, ss_c, sd_c, mx = _mm_scores(x, w, a_s, a_d, bias_in, prelu, r)
    pad = _N_PAD - x.shape[0]
    ss_pad = jnp.pad(ss_c.reshape(-1), (0, pad))
    sd_pad = jnp.pad(sd_c.reshape(-1), (0, pad))
    mx16 = jnp.broadcast_to(mx[0], (16,))
    ex2, den = _edge_ex_den(ss_pad, sd_pad, mx16, sg2, dg2)
    invd = _invden(den)
    return _weighted_scatter(h, ex2.reshape(-1), invd, sg1, dg1, starts,
                             feat)


def kernel(x_protein, x_class, W_a_pos, as_a_pos, ad_a_pos, b_a_pos, W_a_neg, as_a_neg, ad_a_neg, b_a_neg, W_link_rel, W_link_root, b_link, W_a_ppi, as_a_ppi, ad_a_ppi, b_a_ppi, W_b_pos, as_b_pos, ad_b_pos, b_b_pos, W_b_neg, as_b_neg, ad_b_neg, b_b_neg, W_b_ppi, as_b_ppi, ad_b_ppi, b_b_ppi, W_lin, b_lin, edge_index_pos, edge_index_neg, edge_index_link, edge_index_ppi, mask):
    epad = _E_PAD - _E
    s2 = jnp.concatenate([edge_index_ppi[0].astype(jnp.int32),
                          jnp.zeros((epad,), jnp.int32)]).reshape(_EROWS, 128)
    d2 = jnp.concatenate([edge_index_ppi[1].astype(jnp.int32),
                          jnp.full((epad,), _NP, jnp.int32)]
                         ).reshape(_EROWS, 128)

    hist = _bucket_hist(d2)
    offs, starts = _bucket_scan(hist)
    sg, dg = _bucket_place(s2, d2, offs)
    sg2 = sg.reshape(_EROWS, 128)
    dg2 = dg.reshape(_EROWS, 128)

    out_a = _gat_layer(x_protein, W_a_ppi, as_a_ppi, ad_a_ppi,
                       jnp.zeros((_D,), jnp.float32), False, 1000,
                       sg2, dg2, sg, dg, starts, _H)

    out_b = _gat_layer(out_a, W_b_ppi, as_b_ppi, ad_b_ppi,
                       b_a_ppi, True, 784, sg2, dg2, sg, dg, starts, _D)

    npairs = mask.shape[0]
    idx_flat = mask.T.reshape(-1).astype(jnp.int32)
    rows = _pair_gather(out_b, idx_flat, npairs)
    p1 = rows[:npairs]
    p2 = rows[npairs:]
    out = pl.pallas_call(
        _readout_body,
        out_shape=jax.ShapeDtypeStruct((npairs, 1), jnp.float32),
    )(p1, p2, W_lin[:_D], W_lin[_D:], b_b_ppi.reshape(1, _D),
      b_lin.reshape(1, 1))
    return out
